# baseline (speedup 1.0000x reference)
import functools

import jax
import jax.numpy as jnp
from jax import lax
from jax.experimental import pallas as pl
from jax.experimental.pallas import tpu as pltpu
from jax.experimental.pallas import tpu_sc as plsc

T_TOK = 32768
D_DIM = 1024
E_EXP = 8
BT = 2048
THALF = T_TOK // 2

NC, NS, L = 2, 16, 16
NW = NC * NS
CW = THALF // NW


def _route(logits, logits_ref, probs_ref, mask_ref):
    logits_ref[...] = logits
    m1 = jnp.max(logits, axis=0, keepdims=True)
    ex = jnp.exp(logits - m1)
    probs_ref[...] = ex / jnp.sum(ex, axis=0, keepdims=True)
    e = logits.shape[0]
    row = jax.lax.broadcasted_iota(jnp.int32, logits.shape, 0)
    cand1 = jnp.where(logits == m1, row, e)
    i1 = jnp.min(cand1, axis=0, keepdims=True)
    take1 = row == i1
    v2 = jnp.where(take1, -jnp.inf, logits)
    m2 = jnp.max(v2, axis=0, keepdims=True)
    cand2 = jnp.where(v2 == m2, row, e)
    i2 = jnp.min(cand2, axis=0, keepdims=True)
    mask_ref[...] = (take1 | (row == i2)).astype(mask_ref.dtype)


def _mm_body(h_ref, w_ref, logits_ref):
    logits_ref[...] = jax.lax.dot_general(
        w_ref[...], h_ref[...], (((1,), (1,)), ((), ())),
        preferred_element_type=jnp.float32,
    )


def _fused_body(h_ref, w_ref, logits_ref, probs_ref, mask_ref):
    logits = jax.lax.dot_general(
        w_ref[...], h_ref[...], (((1,), (1,)), ((), ())),
        preferred_element_type=jnp.float32,
    )
    _route(logits, logits_ref, probs_ref, mask_ref)


def _tc_mm_half(h, W):
    nblk = THALF // BT
    return pl.pallas_call(
        _mm_body,
        grid=(nblk,),
        in_specs=[
            pl.BlockSpec((BT, D_DIM), lambda i: (i, 0)),
            pl.BlockSpec((E_EXP, D_DIM), lambda i: (0, 0)),
        ],
        out_specs=[pl.BlockSpec((E_EXP, BT), lambda i: (0, i))],
        out_shape=[jax.ShapeDtypeStruct((E_EXP, THALF), jnp.float32)],
        compiler_params=pltpu.CompilerParams(
            dimension_semantics=("arbitrary",),
        ),
    )(h, W)[0]


def _tc_fused_half(h, W):
    nblk = THALF // BT
    return pl.pallas_call(
        _fused_body,
        grid=(nblk,),
        in_specs=[
            pl.BlockSpec((BT, D_DIM), lambda i: (i + nblk, 0)),
            pl.BlockSpec((E_EXP, D_DIM), lambda i: (0, 0)),
        ],
        out_specs=[pl.BlockSpec((E_EXP, BT), lambda i: (0, i))] * 3,
        out_shape=[jax.ShapeDtypeStruct((E_EXP, THALF), jnp.float32)] * 3,
        compiler_params=pltpu.CompilerParams(
            dimension_semantics=("arbitrary",),
        ),
    )(h, W)


def _sc_group(accs):
    m1 = accs[0]
    for e in range(1, E_EXP):
        m1 = jnp.maximum(m1, accs[e])
    ex = [jnp.exp(a - m1) for a in accs]
    s = ex[0]
    for e in range(1, E_EXP):
        s = s + ex[e]
    probs = [x / s for x in ex]

    seen = jnp.zeros((L,), jnp.bool_)
    take1 = []
    for e in range(E_EXP):
        t = (accs[e] == m1) & (~seen)
        take1.append(t)
        seen = seen | t
    v2 = [jnp.where(take1[e], -jnp.inf, accs[e]) for e in range(E_EXP)]
    m2 = v2[0]
    for e in range(1, E_EXP):
        m2 = jnp.maximum(m2, v2[e])
    seen2 = jnp.zeros((L,), jnp.bool_)
    mask = []
    for e in range(E_EXP):
        t2 = (v2[e] == m2) & (~seen2)
        seen2 = seen2 | t2
        mask.append(jnp.where(take1[e] | t2, 1.0, 0.0).astype(jnp.float32))
    return probs, mask


def _sc_body(l_hbm, p_hbm, m_hbm, lbuf, pbuf, mbuf, sem):
    wid = lax.axis_index("s") * NC + lax.axis_index("c")
    wbase = wid * CW

    copy = pltpu.make_async_copy(l_hbm.at[:, pl.ds(wbase, CW)], lbuf, sem)
    copy.start()
    copy.wait()

    def gbody(g, carry):
        col = g * L
        accs = [lbuf[e, pl.ds(col, L)] for e in range(E_EXP)]
        probs, mask = _sc_group(accs)
        for e in range(E_EXP):
            pbuf[e, pl.ds(col, L)] = probs[e]
            mbuf[e, pl.ds(col, L)] = mask[e]
        return carry

    lax.fori_loop(0, CW // L, gbody, 0)

    pltpu.sync_copy(pbuf, p_hbm.at[:, pl.ds(wbase, CW)])
    pltpu.sync_copy(mbuf, m_hbm.at[:, pl.ds(wbase, CW)])


_sc_route = functools.partial(
    pl.kernel,
    out_type=[
        jax.ShapeDtypeStruct((E_EXP, THALF), jnp.float32),
        jax.ShapeDtypeStruct((E_EXP, THALF), jnp.float32),
    ],
    mesh=plsc.VectorSubcoreMesh(core_axis_name="c", subcore_axis_name="s"),
    compiler_params=pltpu.CompilerParams(
        use_tc_tiling_on_sc=False, needs_layout_passes=False
    ),
    scratch_types=[
        pltpu.VMEM((E_EXP, CW), jnp.float32),
        pltpu.VMEM((E_EXP, CW), jnp.float32),
        pltpu.VMEM((E_EXP, CW), jnp.float32),
        pltpu.SemaphoreType.DMA,
    ],
)(_sc_body)


@jax.jit
def kernel(h, W):
    la = _tc_mm_half(h, W)
    lb, pb, mb = _tc_fused_half(h, W)
    pa, ma = _sc_route(la)
    logits = jnp.concatenate([la, lb], axis=1).T
    probs = jnp.concatenate([pa, pb], axis=1).T
    mask = jnp.concatenate([ma, mb], axis=1).T.astype(bool)
    return (mask, probs, logits, logits)


# speedup vs baseline: 1.0056x; 1.0056x over previous
import functools

import jax
import jax.numpy as jnp
from jax import lax
from jax.experimental import pallas as pl
from jax.experimental.pallas import tpu as pltpu
from jax.experimental.pallas import tpu_sc as plsc

T_TOK = 32768
D_DIM = 1024
E_EXP = 8
BT = 2048
THALF = T_TOK // 2

NC, NS, L = 2, 16, 16
NW = NC * NS
CW = THALF // NW


def _route(logits, logits_ref, probs_ref, mask_ref):
    logits_ref[...] = logits
    m1 = jnp.max(logits, axis=0, keepdims=True)
    ex = jnp.exp(logits - m1)
    probs_ref[...] = ex / jnp.sum(ex, axis=0, keepdims=True)
    e = logits.shape[0]
    row = jax.lax.broadcasted_iota(jnp.int32, logits.shape, 0)
    cand1 = jnp.where(logits == m1, row, e)
    i1 = jnp.min(cand1, axis=0, keepdims=True)
    take1 = row == i1
    v2 = jnp.where(take1, -jnp.inf, logits)
    m2 = jnp.max(v2, axis=0, keepdims=True)
    cand2 = jnp.where(v2 == m2, row, e)
    i2 = jnp.min(cand2, axis=0, keepdims=True)
    mask_ref[...] = (take1 | (row == i2)).astype(mask_ref.dtype)


def _mm_body(h_ref, w_ref, logits_ref):
    logits_ref[...] = jax.lax.dot_general(
        w_ref[...], h_ref[...], (((1,), (1,)), ((), ())),
        preferred_element_type=jnp.float32,
    )


def _fused_body(h_ref, w_ref, logits_ref, probs_ref, mask_ref):
    logits = jax.lax.dot_general(
        w_ref[...], h_ref[...], (((1,), (1,)), ((), ())),
        preferred_element_type=jnp.float32,
    )
    _route(logits, logits_ref, probs_ref, mask_ref)


def _tc_mm_half(h, W):
    nblk = THALF // BT
    return pl.pallas_call(
        _mm_body,
        grid=(nblk,),
        in_specs=[
            pl.BlockSpec((BT, D_DIM), lambda i: (i, 0)),
            pl.BlockSpec((E_EXP, D_DIM), lambda i: (0, 0)),
        ],
        out_specs=[pl.BlockSpec((E_EXP, BT), lambda i: (0, i))],
        out_shape=[jax.ShapeDtypeStruct((E_EXP, THALF), jnp.float32)],
        compiler_params=pltpu.CompilerParams(
            dimension_semantics=("arbitrary",),
        ),
    )(h, W)[0]


def _tc_fused_half(h, W):
    nblk = THALF // BT
    return pl.pallas_call(
        _fused_body,
        grid=(nblk,),
        in_specs=[
            pl.BlockSpec((BT, D_DIM), lambda i: (i + nblk, 0)),
            pl.BlockSpec((E_EXP, D_DIM), lambda i: (0, 0)),
        ],
        out_specs=[pl.BlockSpec((E_EXP, BT), lambda i: (0, i))] * 3,
        out_shape=[jax.ShapeDtypeStruct((E_EXP, THALF), jnp.float32)] * 3,
        compiler_params=pltpu.CompilerParams(
            dimension_semantics=("arbitrary",),
        ),
    )(h, W)


def _sc_group(accs):
    m1 = accs[0]
    for e in range(1, E_EXP):
        m1 = jnp.maximum(m1, accs[e])
    ex = [jnp.exp(a - m1) for a in accs]
    s = ex[0]
    for e in range(1, E_EXP):
        s = s + ex[e]
    probs = [x / s for x in ex]

    seen = jnp.zeros((L,), jnp.bool_)
    take1 = []
    for e in range(E_EXP):
        t = (accs[e] == m1) & (~seen)
        take1.append(t)
        seen = seen | t
    v2 = [jnp.where(take1[e], -jnp.inf, accs[e]) for e in range(E_EXP)]
    m2 = v2[0]
    for e in range(1, E_EXP):
        m2 = jnp.maximum(m2, v2[e])
    seen2 = jnp.zeros((L,), jnp.bool_)
    mask = []
    for e in range(E_EXP):
        t2 = (v2[e] == m2) & (~seen2)
        seen2 = seen2 | t2
        mask.append(jnp.where(take1[e] | t2, 1.0, 0.0).astype(jnp.float32))
    return probs, mask


def _sc_body(l_hbm, p_hbm, m_hbm, lbuf, pbuf, mbuf, sem):
    wid = lax.axis_index("s") * NC + lax.axis_index("c")
    wbase = wid * CW

    copy = pltpu.make_async_copy(l_hbm.at[:, pl.ds(wbase, CW)], lbuf, sem)
    copy.start()
    copy.wait()

    def gbody(g, carry):
        col = g * L
        accs = [lbuf[e, pl.ds(col, L)] for e in range(E_EXP)]
        probs, mask = _sc_group(accs)
        for e in range(E_EXP):
            pbuf[e, pl.ds(col, L)] = probs[e]
            mbuf[e, pl.ds(col, L)] = mask[e]
        return carry

    lax.fori_loop(0, CW // L, gbody, 0)

    pltpu.sync_copy(pbuf, p_hbm.at[:, pl.ds(wbase, CW)])
    pltpu.sync_copy(mbuf, m_hbm.at[:, pl.ds(wbase, CW)])


_sc_route = functools.partial(
    pl.kernel,
    out_type=[
        jax.ShapeDtypeStruct((E_EXP, THALF), jnp.float32),
        jax.ShapeDtypeStruct((E_EXP, THALF), jnp.float32),
    ],
    mesh=plsc.VectorSubcoreMesh(core_axis_name="c", subcore_axis_name="s"),
    compiler_params=pltpu.CompilerParams(
        use_tc_tiling_on_sc=False, needs_layout_passes=False
    ),
    scratch_types=[
        pltpu.VMEM((E_EXP, CW), jnp.float32),
        pltpu.VMEM((E_EXP, CW), jnp.float32),
        pltpu.VMEM((E_EXP, CW), jnp.float32),
        pltpu.SemaphoreType.DMA,
    ],
)(_sc_body)


@jax.jit
def kernel(h, W):
    la = _tc_mm_half(h, W)
    pa, ma = _sc_route(la)
    lb, pb, mb = _tc_fused_half(h, W)
    logits = jnp.concatenate([la, lb], axis=1).T
    probs = jnp.concatenate([pa, pb], axis=1).T
    mask = jnp.concatenate([ma, mb], axis=1).T.astype(bool)
    return (mask, probs, logits, logits)


# speedup vs baseline: 1.5173x; 1.5089x over previous
import functools

import jax
import jax.numpy as jnp
from jax.experimental import pallas as pl
from jax.experimental.pallas import tpu as pltpu

BT = 2048


def _router_body(h_ref, w_ref, logits_ref, probs_ref, mask_ref):
    h = h_ref[...]
    w = w_ref[...]
    logits = jax.lax.dot_general(
        w, h, (((1,), (1,)), ((), ())), preferred_element_type=jnp.float32
    )
    logits_ref[...] = logits
    m1 = jnp.max(logits, axis=0, keepdims=True)
    ex = jnp.exp(logits - m1)
    probs_ref[...] = ex / jnp.sum(ex, axis=0, keepdims=True)

    e = logits.shape[0]
    row = jax.lax.broadcasted_iota(jnp.int32, logits.shape, 0)
    cand1 = jnp.where(logits == m1, row, e)
    i1 = jnp.min(cand1, axis=0, keepdims=True)
    take1 = row == i1
    v2 = jnp.where(take1, -jnp.inf, logits)
    m2 = jnp.max(v2, axis=0, keepdims=True)
    cand2 = jnp.where(v2 == m2, row, e)
    i2 = jnp.min(cand2, axis=0, keepdims=True)
    mask_ref[...] = (take1 | (row == i2)).astype(mask_ref.dtype)


@jax.jit
def kernel(h, W):
    t, d = h.shape
    e = W.shape[0]
    grid = (t // BT,)
    logits_t, probs_t, mask_t = pl.pallas_call(
        _router_body,
        grid=grid,
        in_specs=[
            pl.BlockSpec((BT, d), lambda i: (i, 0)),
            pl.BlockSpec((e, d), lambda i: (0, 0)),
        ],
        out_specs=[
            pl.BlockSpec((e, BT), lambda i: (0, i)),
            pl.BlockSpec((e, BT), lambda i: (0, i)),
            pl.BlockSpec((e, BT), lambda i: (0, i)),
        ],
        out_shape=[
            jax.ShapeDtypeStruct((e, t), jnp.float32),
            jax.ShapeDtypeStruct((e, t), jnp.float32),
            jax.ShapeDtypeStruct((e, t), jnp.float32),
        ],
        compiler_params=pltpu.CompilerParams(
            dimension_semantics=("arbitrary",),
        ),
    )(h, W)
    logits = logits_t.T
    return (mask_t.T.astype(bool), probs_t.T, logits, logits)
